# Initial kernel scaffold; baseline (speedup 1.0000x reference)
import jax
import jax.numpy as jnp
from jax.experimental import pallas as pl


def kernel(x, Wkva, Wkvb, Wza, Wzb, b_a, b_b, Wdq, Wiuq, Ww, Wk, Wuq, Wod, Wou, kvn_g, kvn_b, qn_g, qn_b, sink_logits):
    raise NotImplementedError("write your pallas kernel here")


# speedup vs baseline: 8.0698x; 8.0698x over previous
import math

import jax
import jax.numpy as jnp
from jax import lax
from jax.experimental import pallas as pl

F32 = jnp.float32
S = 2048
D = 2048
HD = 128
NH = 16
DC = 512
M_ = 4
TOPK = 256
CI = 64
NHI = 4
NG = 4
DG = 1024
RD = 64
EPS = 1e-6
SK = S // M_
RB = 256
NB = S // RB
NEG = -1e30
WBIG = 4 * HD + DC + CI + NHI + 60


PREC = lax.Precision.HIGHEST
BF16 = jnp.bfloat16


def _dot(a, b):
    return lax.dot_general(a, b, (((1,), (0,)), ((), ())),
                           precision=PREC, preferred_element_type=F32)


def _dot_t(a, b):
    return lax.dot_general(a, b, (((1,), (1,)), ((), ())),
                           precision=PREC, preferred_element_type=F32)


def _dot_bf(a, b):
    return lax.dot_general(a.astype(BF16), b.astype(BF16),
                           (((1,), (0,)), ((), ())),
                           preferred_element_type=F32)


def _dot_t_bf(a, b):
    return lax.dot_general(a.astype(BF16), b.astype(BF16),
                           (((1,), (1,)), ((), ())),
                           preferred_element_type=F32)


def _proj_kernel(x_ref, w4_ref, wdq_ref, o1_ref, o2_ref, o3_ref):
    xx = x_ref[...]
    o1_ref[...] = _dot(xx, w4_ref[...])
    p2 = _dot_bf(xx, wdq_ref[...])
    o2_ref[...] = p2[:, 0:DC]
    o3_ref[...] = p2[:, DC:DC + 128]


def _compress_kernel(cz_ref, xr_ref, wk_ref, ba_ref, bb_ref, g_ref, b_ref,
                     kc_ref, kp_ref):
    cz = cz_ref[...]
    c_a = cz[:, 0:HD]
    c_b = cz[:, HD:2 * HD]
    z_a = cz[:, 2 * HD:3 * HD]
    z_b = cz[:, 3 * HD:4 * HD]
    r4 = lax.broadcasted_iota(jnp.int32, (S, M_), 0)
    j4 = lax.broadcasted_iota(jnp.int32, (S, M_), 1)
    tsel = ((r4 % M_) == j4).astype(F32)
    ba = _dot(tsel, ba_ref[...])
    bb = _dot(tsel, bb_ref[...])
    ea = jnp.exp(z_a + ba)
    eb = jnp.exp(z_b + bb)
    nn = lax.broadcasted_iota(jnp.int32, (SK, S), 0)
    rr = lax.broadcasted_iota(jnp.int32, (SK, S), 1)
    dm = ((rr // M_) == nn).astype(F32)
    ds = ((rr // M_) == (nn - 1)).astype(F32)
    num = _dot(dm, ea * c_a) + _dot(ds, eb * c_b)
    den = _dot(dm, ea) + _dot(ds, eb)
    comp = num / den
    mu = jnp.mean(comp, axis=1, keepdims=True)
    var = jnp.mean((comp - mu) ** 2, axis=1, keepdims=True)
    kc_ref[...] = ((comp - mu) * lax.rsqrt(var + EPS) * g_ref[...]
                   + b_ref[...])
    xr = xr_ref[...]
    ksum = xr[:, 0:D]
    for j in range(1, M_):
        ksum = ksum + xr[:, j * D:(j + 1) * D]
    korig = ksum * (1.0 / M_)
    kp_ref[...] = _dot_bf(korig, wk_ref[...])


def _attn_kernel(cq_ref, misc_ref, kp_ref, kc_ref, wuq_ref, wiuq_ref,
                 qng_ref, qnb_ref, sink_ref, out_ref):
    pid = pl.program_id(0)
    cq = cq_ref[...]
    qi = _dot_bf(cq, wiuq_ref[...])
    kp = kp_ref[...]
    score = jnp.zeros((RB, SK), F32)
    for h in range(NHI):
        dh = _dot_t_bf(qi[:, h * CI:(h + 1) * CI], kp)
        wh = misc_ref[:, h:h + 1]
        score = score + jnp.maximum(dh, 0.0) * wh
    rows = lax.broadcasted_iota(jnp.int32, (RB, SK), 0) + pid * RB
    cols = lax.broadcasted_iota(jnp.int32, (RB, SK), 1)
    score = jnp.where(cols < rows, score, NEG)

    bits = lax.bitcast_convert_type(score, jnp.int32)
    int_min = jnp.int32(-2**31)
    code = jnp.where(bits >= 0, bits + int_min, -bits)

    def body(i, carry):
        sel, cand, krem = carry
        b = 31 - i
        bit = (lax.shift_right_logical(code, b) & 1).astype(F32)
        ones = cand * bit
        c1 = jnp.sum(ones, axis=1, keepdims=True)
        ta = (c1 < krem).astype(F32)
        sel = sel + ones * ta
        krem = krem - c1 * ta
        cand = ones + ta * (cand - 2.0 * ones)
        return sel, cand, krem

    sel0 = jnp.zeros((RB, SK), F32)
    cand0 = jnp.ones((RB, SK), F32)
    krem0 = jnp.full((RB, 1), float(TOPK), F32)
    sel, cand, krem = lax.fori_loop(0, 32, body, (sel0, cand0, krem0))
    li = lax.broadcasted_iota(jnp.int32, (SK, SK), 0)
    lj = lax.broadcasted_iota(jnp.int32, (SK, SK), 1)
    lt = (li < lj).astype(F32)
    eqr = _dot(cand, lt)
    sel = sel + cand * (eqr < krem).astype(F32)

    keep = (sel > 0.5) & (rows < cols * M_)
    abias = jnp.where(keep, 0.0, NEG)

    qall = _dot(cq, wuq_ref[...])
    kc = kc_ref[...]
    lidx = lax.broadcasted_iota(jnp.int32, (RB, HD), 1)
    srow = (lax.broadcasted_iota(jnp.int32, (RB, HD), 0) + pid * RB
            ).astype(F32)
    expo = ((lidx - RD) & (~1)).astype(F32) * (1.0 / RD)
    theta = jnp.exp(expo * (-math.log(10000.0)))
    ang = srow * theta
    cosf = jnp.where(lidx < RD, 1.0, jnp.cos(ang))
    sinf = jnp.where(lidx < RD, 0.0,
                     jnp.where((lidx & 1) == 0, -jnp.sin(ang), jnp.sin(ang)))
    pr = lax.broadcasted_iota(jnp.int32, (HD, HD), 0)
    pc = lax.broadcasted_iota(jnp.int32, (HD, HD), 1)
    perm = (pr == (pc ^ 1)).astype(F32)
    inv = 1.0 / math.sqrt(float(HD))
    sink = jnp.exp(sink_ref[...])
    for h in range(NH):
        qh = qall[:, h * HD:(h + 1) * HD]
        mu = jnp.mean(qh, axis=1, keepdims=True)
        var = jnp.mean((qh - mu) ** 2, axis=1, keepdims=True)
        qh = (qh - mu) * lax.rsqrt(var + EPS) * qng_ref[...] + qnb_ref[...]
        qh = qh * cosf + _dot(qh, perm) * sinf
        sc_h = _dot_t(qh, kc) * inv + abias
        es = jnp.exp(sc_h)
        den = jnp.sum(es, axis=1, keepdims=True) + sink[:, h:h + 1]
        out_ref[:, h * HD:(h + 1) * HD] = _dot(es / den, kc)


def _out_kernel(o_ref, wod_ref, wou_ref, f_ref):
    o = o_ref[...]
    acc = jnp.zeros((RB, D), F32)
    gi = (NH * HD) // NG
    for g in range(NG):
        dg = _dot(o[:, g * gi:(g + 1) * gi],
                  wod_ref[g * gi:(g + 1) * gi, :])
        acc = acc + _dot(dg, wou_ref[g * DG:(g + 1) * DG, :])
    f_ref[...] = acc


def kernel(x, Wkva, Wkvb, Wza, Wzb, b_a, b_b, Wdq, Wiuq, Ww, Wk, Wuq,
           Wod, Wou, kvn_g, kvn_b, qn_g, qn_b, sink_logits):
    x2 = x.reshape(S, D)
    w4 = jnp.concatenate([Wkva, Wkvb, Wza, Wzb], axis=1)
    wdqw = jnp.concatenate([Wdq, Ww, jnp.zeros((D, 124), F32)], axis=1)

    o1, o2, o3 = pl.pallas_call(
        _proj_kernel,
        grid=(NB,),
        in_specs=[pl.BlockSpec((RB, D), lambda i: (i, 0)),
                  pl.BlockSpec((D, 4 * HD), lambda i: (0, 0)),
                  pl.BlockSpec((D, DC + 128), lambda i: (0, 0))],
        out_specs=[pl.BlockSpec((RB, 4 * HD), lambda i: (i, 0)),
                   pl.BlockSpec((RB, DC), lambda i: (i, 0)),
                   pl.BlockSpec((RB, 128), lambda i: (i, 0))],
        out_shape=[jax.ShapeDtypeStruct((S, 4 * HD), F32),
                   jax.ShapeDtypeStruct((S, DC), F32),
                   jax.ShapeDtypeStruct((S, 128), F32)],
    )(x2, w4, wdqw)

    kc, kp = pl.pallas_call(
        _compress_kernel,
        in_specs=[pl.BlockSpec((S, 4 * HD), lambda: (0, 0)),
                  pl.BlockSpec((SK, M_ * D), lambda: (0, 0)),
                  pl.BlockSpec((D, CI), lambda: (0, 0)),
                  pl.BlockSpec((M_, HD), lambda: (0, 0)),
                  pl.BlockSpec((M_, HD), lambda: (0, 0)),
                  pl.BlockSpec((1, HD), lambda: (0, 0)),
                  pl.BlockSpec((1, HD), lambda: (0, 0))],
        out_specs=[pl.BlockSpec((SK, HD), lambda: (0, 0)),
                   pl.BlockSpec((SK, CI), lambda: (0, 0))],
        out_shape=[jax.ShapeDtypeStruct((SK, HD), F32),
                   jax.ShapeDtypeStruct((SK, CI), F32)],
    )(o1, x.reshape(SK, M_ * D), Wk, b_a, b_b,
      kvn_g.reshape(1, HD), kvn_b.reshape(1, HD))

    att = pl.pallas_call(
        _attn_kernel,
        grid=(NB,),
        in_specs=[pl.BlockSpec((RB, DC), lambda i: (i, 0)),
                  pl.BlockSpec((RB, 128), lambda i: (i, 0)),
                  pl.BlockSpec((SK, CI), lambda i: (0, 0)),
                  pl.BlockSpec((SK, HD), lambda i: (0, 0)),
                  pl.BlockSpec((DC, NH * HD), lambda i: (0, 0)),
                  pl.BlockSpec((DC, NHI * CI), lambda i: (0, 0)),
                  pl.BlockSpec((1, HD), lambda i: (0, 0)),
                  pl.BlockSpec((1, HD), lambda i: (0, 0)),
                  pl.BlockSpec((1, NH), lambda i: (0, 0))],
        out_specs=pl.BlockSpec((RB, NH * HD), lambda i: (i, 0)),
        out_shape=jax.ShapeDtypeStruct((S, NH * HD), F32),
    )(o2, o3, kp, kc, Wuq, Wiuq, qn_g.reshape(1, HD), qn_b.reshape(1, HD),
      sink_logits.reshape(1, NH))

    fin = pl.pallas_call(
        _out_kernel,
        grid=(NB,),
        in_specs=[pl.BlockSpec((RB, NH * HD), lambda i: (i, 0)),
                  pl.BlockSpec((NG * 512, DG), lambda i: (0, 0)),
                  pl.BlockSpec((NG * DG, D), lambda i: (0, 0))],
        out_specs=pl.BlockSpec((RB, D), lambda i: (i, 0)),
        out_shape=jax.ShapeDtypeStruct((S, D), F32),
    )(att, Wod.reshape(NG * 512, DG), Wou)

    return fin.reshape(1, S, D)


# speedup vs baseline: 12.9959x; 1.6104x over previous
import math

import jax
import jax.numpy as jnp
from jax import lax
from jax.experimental import pallas as pl
from jax.experimental.pallas import tpu as pltpu

F32 = jnp.float32
S = 2048
D = 2048
HD = 128
NH = 16
DC = 512
M_ = 4
TOPK = 256
CI = 64
NHI = 4
NG = 4
DG = 1024
RD = 64
EPS = 1e-6
SK = S // M_
RB = 256
NB = S // RB
NEG = -1e30
WBIG = 4 * HD + DC + CI + NHI + 60


PREC = lax.Precision.HIGHEST
BF16 = jnp.bfloat16


def _dot(a, b):
    return lax.dot_general(a, b, (((1,), (0,)), ((), ())),
                           precision=PREC, preferred_element_type=F32)


def _dot_t(a, b):
    return lax.dot_general(a, b, (((1,), (1,)), ((), ())),
                           precision=PREC, preferred_element_type=F32)


def _split_bf(a):
    hi = a.astype(BF16)
    lo = (a - hi.astype(F32)).astype(BF16)
    return hi, lo


def _dot_h(a, b):
    dims = (((1,), (0,)), ((), ()))
    ah, al = _split_bf(a)
    bh, bl = _split_bf(b)
    d = lax.dot_general(ah, bh, dims, preferred_element_type=F32)
    d = d + lax.dot_general(ah, bl, dims, preferred_element_type=F32)
    d = d + lax.dot_general(al, bh, dims, preferred_element_type=F32)
    return d


def _dot_t_h(a, b):
    dims = (((1,), (1,)), ((), ()))
    ah, al = _split_bf(a)
    bh, bl = _split_bf(b)
    d = lax.dot_general(ah, bh, dims, preferred_element_type=F32)
    d = d + lax.dot_general(ah, bl, dims, preferred_element_type=F32)
    d = d + lax.dot_general(al, bh, dims, preferred_element_type=F32)
    return d


def _dot_bf(a, b):
    return lax.dot_general(a.astype(BF16), b.astype(BF16),
                           (((1,), (0,)), ((), ())),
                           preferred_element_type=F32)


def _dot_t_bf(a, b):
    return lax.dot_general(a.astype(BF16), b.astype(BF16),
                           (((1,), (1,)), ((), ())),
                           preferred_element_type=F32)


def _proj_kernel(x_ref, w4_ref, wdq_ref, o1_ref, o2_ref, o3_ref):
    xx = x_ref[...]
    o1_ref[...] = _dot_h(xx, w4_ref[...])
    p2 = _dot_bf(xx, wdq_ref[...])
    o2_ref[...] = p2[:, 0:DC]
    o3_ref[...] = p2[:, DC:DC + 128]


def _compress_kernel(cz_ref, xr_ref, wk_ref, ba_ref, bb_ref, g_ref, b_ref,
                     kc_ref, kp_ref):
    cz = cz_ref[...]
    c_a = cz[:, 0:HD]
    c_b = cz[:, HD:2 * HD]
    z_a = cz[:, 2 * HD:3 * HD]
    z_b = cz[:, 3 * HD:4 * HD]
    r4 = lax.broadcasted_iota(jnp.int32, (S, M_), 0)
    j4 = lax.broadcasted_iota(jnp.int32, (S, M_), 1)
    tsel = ((r4 % M_) == j4).astype(F32)
    ba = _dot(tsel, ba_ref[...])
    bb = _dot(tsel, bb_ref[...])
    ea = jnp.exp(z_a + ba)
    eb = jnp.exp(z_b + bb)
    nn = lax.broadcasted_iota(jnp.int32, (SK, S), 0)
    rr = lax.broadcasted_iota(jnp.int32, (SK, S), 1)
    dm = ((rr // M_) == nn).astype(F32)
    ds = ((rr // M_) == (nn - 1)).astype(F32)
    num = _dot(dm, ea * c_a) + _dot(ds, eb * c_b)
    den = _dot(dm, ea) + _dot(ds, eb)
    comp = num / den
    mu = jnp.mean(comp, axis=1, keepdims=True)
    var = jnp.mean((comp - mu) ** 2, axis=1, keepdims=True)
    kc_ref[...] = ((comp - mu) * lax.rsqrt(var + EPS) * g_ref[...]
                   + b_ref[...])
    xr = xr_ref[...]
    ksum = xr[:, 0:D]
    for j in range(1, M_):
        ksum = ksum + xr[:, j * D:(j + 1) * D]
    korig = ksum * (1.0 / M_)
    kp_ref[...] = _dot_bf(korig, wk_ref[...])


def _attn_kernel(cq_ref, misc_ref, kp_ref, kc_ref, wuq_ref, wiuq_ref,
                 qng_ref, qnb_ref, sink_ref, out_ref):
    pid = pl.program_id(0)
    cq = cq_ref[...]
    qi = _dot_bf(cq, wiuq_ref[...])
    kp = kp_ref[...]
    score = jnp.zeros((RB, SK), F32)
    for h in range(NHI):
        dh = _dot_t_bf(qi[:, h * CI:(h + 1) * CI], kp)
        wh = misc_ref[:, h:h + 1]
        score = score + jnp.maximum(dh, 0.0) * wh
    rows = lax.broadcasted_iota(jnp.int32, (RB, SK), 0) + pid * RB
    cols = lax.broadcasted_iota(jnp.int32, (RB, SK), 1)
    score = jnp.where(cols < rows, score, NEG)

    bits = lax.bitcast_convert_type(score, jnp.int32)
    int_min = jnp.int32(-2**31)
    code = jnp.where(bits >= 0, bits + int_min, -bits)

    def body(i, carry):
        sel, cand, krem = carry
        b = 31 - i
        bit = (lax.shift_right_logical(code, b) & 1).astype(F32)
        ones = cand * bit
        c1 = jnp.sum(ones, axis=1, keepdims=True)
        ta = (c1 < krem).astype(F32)
        sel = sel + ones * ta
        krem = krem - c1 * ta
        cand = ones + ta * (cand - 2.0 * ones)
        return sel, cand, krem

    sel0 = jnp.zeros((RB, SK), F32)
    cand0 = jnp.ones((RB, SK), F32)
    krem0 = jnp.full((RB, 1), float(TOPK), F32)
    sel, cand, krem = lax.fori_loop(0, 32, body, (sel0, cand0, krem0))
    li = lax.broadcasted_iota(jnp.int32, (SK, SK), 0)
    lj = lax.broadcasted_iota(jnp.int32, (SK, SK), 1)
    lt = (li < lj).astype(F32)
    eqr = _dot_bf(cand, lt)
    sel = sel + cand * (eqr < krem).astype(F32)

    keep = (sel > 0.5) & (rows < cols * M_)
    abias = jnp.where(keep, 0.0, NEG)

    qall = _dot_h(cq, wuq_ref[...])
    kc = kc_ref[...]
    lidx = lax.broadcasted_iota(jnp.int32, (RB, HD), 1)
    srow = (lax.broadcasted_iota(jnp.int32, (RB, HD), 0) + pid * RB
            ).astype(F32)
    expo = ((lidx - RD) & (~1)).astype(F32) * (1.0 / RD)
    theta = jnp.exp(expo * (-math.log(10000.0)))
    ang = srow * theta
    cosf = jnp.where(lidx < RD, 1.0, jnp.cos(ang))
    sinf = jnp.where(lidx < RD, 0.0,
                     jnp.where((lidx & 1) == 0, -jnp.sin(ang), jnp.sin(ang)))
    even = (lidx & 1) == 0
    inv = 1.0 / math.sqrt(float(HD))
    sink = jnp.exp(sink_ref[...])
    for h in range(NH):
        qh = qall[:, h * HD:(h + 1) * HD]
        mu = jnp.mean(qh, axis=1, keepdims=True)
        var = jnp.mean((qh - mu) ** 2, axis=1, keepdims=True)
        qh = (qh - mu) * lax.rsqrt(var + EPS) * qng_ref[...] + qnb_ref[...]
        qsw = jnp.where(even, pltpu.roll(qh, HD - 1, 1),
                        pltpu.roll(qh, 1, 1))
        qh = qh * cosf + qsw * sinf
        sc_h = _dot_t_h(qh, kc) * inv + abias
        es = jnp.exp(sc_h)
        den = jnp.sum(es, axis=1, keepdims=True) + sink[:, h:h + 1]
        out_ref[:, h * HD:(h + 1) * HD] = _dot_h(es / den, kc)


def _out_kernel(o_ref, wod_ref, wou_ref, f_ref):
    o = o_ref[...]
    acc = jnp.zeros((RB, D), F32)
    gi = (NH * HD) // NG
    for g in range(NG):
        dg = _dot_h(o[:, g * gi:(g + 1) * gi],
                    wod_ref[g * gi:(g + 1) * gi, :])
        acc = acc + _dot_h(dg, wou_ref[g * DG:(g + 1) * DG, :])
    f_ref[...] = acc


def kernel(x, Wkva, Wkvb, Wza, Wzb, b_a, b_b, Wdq, Wiuq, Ww, Wk, Wuq,
           Wod, Wou, kvn_g, kvn_b, qn_g, qn_b, sink_logits):
    x2 = x.reshape(S, D)
    w4 = jnp.concatenate([Wkva, Wkvb, Wza, Wzb], axis=1)
    wdqw = jnp.concatenate([Wdq, Ww, jnp.zeros((D, 124), F32)], axis=1)

    o1, o2, o3 = pl.pallas_call(
        _proj_kernel,
        grid=(NB,),
        in_specs=[pl.BlockSpec((RB, D), lambda i: (i, 0)),
                  pl.BlockSpec((D, 4 * HD), lambda i: (0, 0)),
                  pl.BlockSpec((D, DC + 128), lambda i: (0, 0))],
        out_specs=[pl.BlockSpec((RB, 4 * HD), lambda i: (i, 0)),
                   pl.BlockSpec((RB, DC), lambda i: (i, 0)),
                   pl.BlockSpec((RB, 128), lambda i: (i, 0))],
        out_shape=[jax.ShapeDtypeStruct((S, 4 * HD), F32),
                   jax.ShapeDtypeStruct((S, DC), F32),
                   jax.ShapeDtypeStruct((S, 128), F32)],
    )(x2, w4, wdqw)

    kc, kp = pl.pallas_call(
        _compress_kernel,
        in_specs=[pl.BlockSpec((S, 4 * HD), lambda: (0, 0)),
                  pl.BlockSpec((SK, M_ * D), lambda: (0, 0)),
                  pl.BlockSpec((D, CI), lambda: (0, 0)),
                  pl.BlockSpec((M_, HD), lambda: (0, 0)),
                  pl.BlockSpec((M_, HD), lambda: (0, 0)),
                  pl.BlockSpec((1, HD), lambda: (0, 0)),
                  pl.BlockSpec((1, HD), lambda: (0, 0))],
        out_specs=[pl.BlockSpec((SK, HD), lambda: (0, 0)),
                   pl.BlockSpec((SK, CI), lambda: (0, 0))],
        out_shape=[jax.ShapeDtypeStruct((SK, HD), F32),
                   jax.ShapeDtypeStruct((SK, CI), F32)],
    )(o1, x.reshape(SK, M_ * D), Wk, b_a, b_b,
      kvn_g.reshape(1, HD), kvn_b.reshape(1, HD))

    att = pl.pallas_call(
        _attn_kernel,
        grid=(NB,),
        in_specs=[pl.BlockSpec((RB, DC), lambda i: (i, 0)),
                  pl.BlockSpec((RB, 128), lambda i: (i, 0)),
                  pl.BlockSpec((SK, CI), lambda i: (0, 0)),
                  pl.BlockSpec((SK, HD), lambda i: (0, 0)),
                  pl.BlockSpec((DC, NH * HD), lambda i: (0, 0)),
                  pl.BlockSpec((DC, NHI * CI), lambda i: (0, 0)),
                  pl.BlockSpec((1, HD), lambda i: (0, 0)),
                  pl.BlockSpec((1, HD), lambda i: (0, 0)),
                  pl.BlockSpec((1, NH), lambda i: (0, 0))],
        out_specs=pl.BlockSpec((RB, NH * HD), lambda i: (i, 0)),
        out_shape=jax.ShapeDtypeStruct((S, NH * HD), F32),
    )(o2, o3, kp, kc, Wuq, Wiuq, qn_g.reshape(1, HD), qn_b.reshape(1, HD),
      sink_logits.reshape(1, NH))

    fin = pl.pallas_call(
        _out_kernel,
        grid=(NB,),
        in_specs=[pl.BlockSpec((RB, NH * HD), lambda i: (i, 0)),
                  pl.BlockSpec((NG * 512, DG), lambda i: (0, 0)),
                  pl.BlockSpec((NG * DG, D), lambda i: (0, 0))],
        out_specs=pl.BlockSpec((RB, D), lambda i: (i, 0)),
        out_shape=jax.ShapeDtypeStruct((S, D), F32),
    )(att, Wod.reshape(NG * 512, DG), Wou)

    return fin.reshape(1, S, D)
